# baseline (speedup 1.0000x reference)
import numpy as np
import jax
import jax.numpy as jnp
from jax.experimental import pallas as pl
from jax.experimental.pallas import tpu as pltpu

_MATCH_IOU = 0.5
_IGNORE_IOU = 0.4
_BACKGROUND = -1.0
_IGNORE = -2.0

_BLK = 8192


def _anchor_table(image_size, blk):
    anchors = []
    scales = [2.0 ** 0, 2.0 ** (1.0 / 3.0), 2.0 ** (2.0 / 3.0)]
    ratios = [0.5, 1.0, 2.0]
    for level in range(3, 8):
        stride = 2 ** level
        feat = image_size // stride
        cx = (np.arange(feat, dtype=np.float32) + 0.5) * stride
        cy = (np.arange(feat, dtype=np.float32) + 0.5) * stride
        cxg, cyg = np.meshgrid(cx, cy)
        centers = np.stack([cxg.ravel(), cyg.ravel()], axis=-1)
        base = 4.0 * stride
        sizes = []
        for scale in scales:
            for ratio in ratios:
                area = (base * scale) ** 2
                w = np.sqrt(area / ratio)
                h = w * ratio
                sizes.append([w, h])
        sizes = np.array(sizes, dtype=np.float32)
        c = np.repeat(centers, sizes.shape[0], axis=0)
        s = np.tile(sizes, (centers.shape[0], 1))
        xy = (c - s / 2.0).astype(np.float32)
        anchors.append(np.concatenate([xy, s], axis=-1).astype(np.float32))
    a = np.concatenate(anchors, axis=0)
    m = a.shape[0]
    mpad = ((m + blk - 1) // blk) * blk
    pad = np.zeros((mpad - m, 4), dtype=np.float32)
    pad[:, 2:4] = 1.0
    a = np.concatenate([a, pad], axis=0)
    xy = a[:, 0:2]
    wh = a[:, 2:4]
    x2y2 = xy + wh
    cxy = xy + wh / np.float32(2.0)
    area = wh[:, 0:1] * wh[:, 1:2]
    tab = np.concatenate([xy, x2y2, cxy, wh, area], axis=-1).astype(np.float32)
    return np.ascontiguousarray(tab.T), m


def _encode_body(gtb_ref, gt_t_ref, var_ref, anc_ref, out_ref):
    anc = anc_ref[...]
    gtb = gtb_ref[0]
    n = gtb.shape[0]
    blk = anc.shape[1]

    g1x = gtb[:, 0:1]
    g1y = gtb[:, 1:2]
    g2x = gtb[:, 2:3]
    g2y = gtb[:, 3:4]
    garea = gtb[:, 4:5]

    ax1 = anc[0:1, :]
    ay1 = anc[1:2, :]
    ax2 = anc[2:3, :]
    ay2 = anc[3:4, :]
    aarea = anc[8:9, :]

    iw = jnp.maximum(jnp.minimum(ax2, g2x) - jnp.maximum(ax1, g1x), 0.0)
    ih = jnp.maximum(jnp.minimum(ay2, g2y) - jnp.maximum(ay1, g1y), 0.0)
    inter = iw * ih
    union = (aarea + garea) - inter
    iou = inter / jnp.maximum(union, 1e-8)

    max_iou = jnp.max(iou, axis=0, keepdims=True)
    row = jax.lax.broadcasted_iota(jnp.int32, (n, blk), 0)
    idx = jnp.min(jnp.where(iou == max_iou, row, n), axis=0, keepdims=True)
    onehot = (row == idx).astype(jnp.float32)
    matched = jnp.dot(gt_t_ref[0], onehot,
                      preferred_element_type=jnp.float32)

    m_x = matched[0:1, :]
    m_y = matched[1:2, :]
    m_w = matched[2:3, :]
    m_h = matched[3:4, :]
    m_cls = matched[4:5, :]
    acx = anc[4:5, :]
    acy = anc[5:6, :]
    aw = anc[6:7, :]
    ah = anc[7:8, :]

    v0 = var_ref[0]
    v1 = var_ref[1]
    v2 = var_ref[2]
    v3 = var_ref[3]
    tx = (m_x + m_w / 2.0 - acx) / aw / v0
    ty = (m_y + m_h / 2.0 - acy) / ah / v1
    tw = jnp.log(m_w / aw) / v2
    th = jnp.log(m_h / ah) / v3

    positive = max_iou >= _MATCH_IOU
    negative = max_iou < _IGNORE_IOU
    ignore = jnp.logical_not(jnp.logical_or(positive, negative))
    cls_t = jnp.where(jnp.logical_not(positive), _BACKGROUND, m_cls)
    cls_t = jnp.where(ignore, _IGNORE, cls_t)

    nan_mask = (jnp.isnan(tx) | jnp.isnan(ty) | jnp.isnan(tw)
                | jnp.isnan(th) | jnp.isnan(cls_t))
    tx = jnp.where(nan_mask, _IGNORE, tx)
    ty = jnp.where(nan_mask, _IGNORE, ty)
    tw = jnp.where(nan_mask, _IGNORE, tw)
    th = jnp.where(nan_mask, _IGNORE, th)
    cls_t = jnp.where(nan_mask, _IGNORE, cls_t)

    lab = jnp.concatenate(
        [tx, ty, tw, th, cls_t, cls_t, cls_t, cls_t], axis=0)
    lab_t = jnp.transpose(lab, (1, 0))
    out_ref[0] = lab_t[:, 0:5]


def kernel(images, target_boxes, box_variance):
    b, n, _ = target_boxes.shape
    anc_np, m = _anchor_table(images.shape[1], _BLK)
    mpad = anc_np.shape[1]
    anc = jnp.asarray(anc_np)

    gt_t = jnp.transpose(target_boxes, (0, 2, 1))
    xy = target_boxes[..., 0:2]
    wh = target_boxes[..., 2:4]
    gtb = jnp.concatenate(
        [xy, xy + wh, wh[..., 0:1] * wh[..., 1:2]], axis=-1)

    out = pl.pallas_call(
        _encode_body,
        grid=(mpad // _BLK, b),
        in_specs=[
            pl.BlockSpec((1, n, 5), lambda j, i: (i, 0, 0)),
            pl.BlockSpec((1, 5, n), lambda j, i: (i, 0, 0)),
            pl.BlockSpec(memory_space=pltpu.SMEM),
            pl.BlockSpec((9, _BLK), lambda j, i: (0, j)),
        ],
        out_specs=pl.BlockSpec((1, _BLK, 5), lambda j, i: (i, j, 0)),
        out_shape=jax.ShapeDtypeStruct((b, m, 5), jnp.float32),
        compiler_params=pltpu.CompilerParams(
            dimension_semantics=("parallel", "parallel"),
        ),
    )(gtb, gt_t, box_variance, anc)

    return out


# speedup vs baseline: 1.4722x; 1.4722x over previous
import numpy as np
import jax
import jax.numpy as jnp
from jax.experimental import pallas as pl
from jax.experimental.pallas import tpu as pltpu

_MATCH_IOU = 0.5
_IGNORE_IOU = 0.4
_BACKGROUND = -1.0
_IGNORE = -2.0

_BLK = 8192


def _anchor_table(image_size, blk):
    anchors = []
    scales = [2.0 ** 0, 2.0 ** (1.0 / 3.0), 2.0 ** (2.0 / 3.0)]
    ratios = [0.5, 1.0, 2.0]
    for level in range(3, 8):
        stride = 2 ** level
        feat = image_size // stride
        cx = (np.arange(feat, dtype=np.float32) + 0.5) * stride
        cy = (np.arange(feat, dtype=np.float32) + 0.5) * stride
        cxg, cyg = np.meshgrid(cx, cy)
        centers = np.stack([cxg.ravel(), cyg.ravel()], axis=-1)
        base = 4.0 * stride
        sizes = []
        for scale in scales:
            for ratio in ratios:
                area = (base * scale) ** 2
                w = np.sqrt(area / ratio)
                h = w * ratio
                sizes.append([w, h])
        sizes = np.array(sizes, dtype=np.float32)
        c = np.repeat(centers, sizes.shape[0], axis=0)
        s = np.tile(sizes, (centers.shape[0], 1))
        xy = (c - s / 2.0).astype(np.float32)
        anchors.append(np.concatenate([xy, s], axis=-1).astype(np.float32))
    a = np.concatenate(anchors, axis=0)
    m = a.shape[0]
    mpad = ((m + blk - 1) // blk) * blk
    pad = np.zeros((mpad - m, 4), dtype=np.float32)
    pad[:, 2:4] = 1.0
    a = np.concatenate([a, pad], axis=0)
    xy = a[:, 0:2]
    wh = a[:, 2:4]
    x2y2 = xy + wh
    cxy = xy + wh / np.float32(2.0)
    area = wh[:, 0:1] * wh[:, 1:2]
    tab = np.concatenate([xy, x2y2, cxy, wh, area], axis=-1).astype(np.float32)
    return np.ascontiguousarray(tab.T), m


def _encode_body(gtb_ref, gt_t_ref, var_ref, anc_ref, out_ref):
    anc = anc_ref[...]
    gtb = gtb_ref[0]
    n = gtb.shape[0]
    blk = anc.shape[1]

    g1x = gtb[:, 0:1]
    g1y = gtb[:, 1:2]
    g2x = gtb[:, 2:3]
    g2y = gtb[:, 3:4]
    garea = gtb[:, 4:5]

    ax1 = anc[0:1, :]
    ay1 = anc[1:2, :]
    ax2 = anc[2:3, :]
    ay2 = anc[3:4, :]
    aarea = anc[8:9, :]

    iw = jnp.maximum(jnp.minimum(ax2, g2x) - jnp.maximum(ax1, g1x), 0.0)
    ih = jnp.maximum(jnp.minimum(ay2, g2y) - jnp.maximum(ay1, g1y), 0.0)
    inter = iw * ih
    union = (aarea + garea) - inter
    iou = inter / jnp.maximum(union, 1e-8)

    max_iou = jnp.max(iou, axis=0, keepdims=True)
    row = jax.lax.broadcasted_iota(jnp.int32, (n, blk), 0)
    idx = jnp.min(jnp.where(iou == max_iou, row, n), axis=0, keepdims=True)
    onehot = (row == idx).astype(jnp.float32)
    matched = jnp.dot(gt_t_ref[0], onehot,
                      preferred_element_type=jnp.float32)

    m_x = matched[0:1, :]
    m_y = matched[1:2, :]
    m_w = matched[2:3, :]
    m_h = matched[3:4, :]
    m_cls = matched[4:5, :]
    acx = anc[4:5, :]
    acy = anc[5:6, :]
    aw = anc[6:7, :]
    ah = anc[7:8, :]

    v0 = var_ref[0]
    v1 = var_ref[1]
    v2 = var_ref[2]
    v3 = var_ref[3]
    tx = (m_x + m_w / 2.0 - acx) / aw / v0
    ty = (m_y + m_h / 2.0 - acy) / ah / v1
    tw = jnp.log(m_w / aw) / v2
    th = jnp.log(m_h / ah) / v3

    positive = max_iou >= _MATCH_IOU
    negative = max_iou < _IGNORE_IOU
    ignore = jnp.logical_not(jnp.logical_or(positive, negative))
    cls_t = jnp.where(jnp.logical_not(positive), _BACKGROUND, m_cls)
    cls_t = jnp.where(ignore, _IGNORE, cls_t)

    nan_mask = (jnp.isnan(tx) | jnp.isnan(ty) | jnp.isnan(tw)
                | jnp.isnan(th) | jnp.isnan(cls_t))
    tx = jnp.where(nan_mask, _IGNORE, tx)
    ty = jnp.where(nan_mask, _IGNORE, ty)
    tw = jnp.where(nan_mask, _IGNORE, tw)
    th = jnp.where(nan_mask, _IGNORE, th)
    cls_t = jnp.where(nan_mask, _IGNORE, cls_t)

    out_ref[0, 0:1, :] = tx
    out_ref[0, 1:2, :] = ty
    out_ref[0, 2:3, :] = tw
    out_ref[0, 3:4, :] = th
    out_ref[0, 4:5, :] = cls_t


def kernel(images, target_boxes, box_variance):
    b, n, _ = target_boxes.shape
    anc_np, m = _anchor_table(images.shape[1], _BLK)
    mpad = anc_np.shape[1]
    anc = jnp.asarray(anc_np)

    gt_t = jnp.transpose(target_boxes, (0, 2, 1))
    xy = target_boxes[..., 0:2]
    wh = target_boxes[..., 2:4]
    gtb = jnp.concatenate(
        [xy, xy + wh, wh[..., 0:1] * wh[..., 1:2]], axis=-1)

    out = pl.pallas_call(
        _encode_body,
        grid=(mpad // _BLK, b),
        in_specs=[
            pl.BlockSpec((1, n, 5), lambda j, i: (i, 0, 0)),
            pl.BlockSpec((1, 5, n), lambda j, i: (i, 0, 0)),
            pl.BlockSpec(memory_space=pltpu.SMEM),
            pl.BlockSpec((9, _BLK), lambda j, i: (0, j)),
        ],
        out_specs=pl.BlockSpec((1, 5, _BLK), lambda j, i: (i, 0, j)),
        out_shape=jax.ShapeDtypeStruct((b, 5, mpad), jnp.float32),
        compiler_params=pltpu.CompilerParams(
            dimension_semantics=("parallel", "parallel"),
        ),
    )(gtb, gt_t, box_variance, anc)

    return jnp.transpose(out[:, :, :m], (0, 2, 1))
